# baseline (speedup 1.0000x reference)
import functools

import jax
import jax.numpy as jnp
from jax import lax
from jax.experimental import pallas as pl
from jax.experimental.pallas import tpu as pltpu
from jax.experimental.pallas import tpu_sc as plsc

_K = 20
_EPS = 1e-12


def _sc_pack_body(P, C, NC, TOT,
                  ql_hbm, cs_hbm, qf_hbm, packed_hbm, m_hbm,
                  ql_v, cs_v, idxfg_v, idxbg_v, gidx_v, vals_v, m_v, sem):
    wid = lax.axis_index("s") * NC + lax.axis_index("c")
    pltpu.sync_copy(ql_hbm, ql_v)
    pltpu.sync_copy(cs_hbm, cs_v)
    c0 = cs_v[0, :]
    c1 = cs_v[1, :]
    c2 = cs_v[2, :]
    iota = lax.iota(jnp.int32, 16)

    def mstep(k, carry):
        ofg, obg = carry
        a = ql_v[0, pl.ds(k * 16, 16)]
        b = ql_v[1, pl.ds(k * 16, 16)]
        e = ql_v[2, pl.ds(k * 16, 16)]
        mk = (a == c0) & (b == c1) & (e == c2)
        pvec = k * 16 + iota
        plsc.store_compressed(idxfg_v.at[pl.ds(ofg, 16)], pvec, mask=mk)
        plsc.store_compressed(idxbg_v.at[pl.ds(obg, 16)], pvec,
                              mask=jnp.logical_not(mk))
        cnt = plsc.all_reduce_population_count(mk)[0]
        return (ofg + cnt, obg + (16 - cnt))

    M, Mb = lax.fori_loop(0, P // 16, mstep, (jnp.int32(0), jnp.int32(0)))

    fg_total = C * M
    msafe = jnp.maximum(M, 1)
    mbsafe = jnp.maximum(Mb, 1)
    base = wid * TOT

    def gstep(k, carry):
        i = base + k * 16 + iota
        isfg = i < fg_total
        cf = lax.div(i, msafe)
        jf = i - cf * msafe
        ib = jnp.maximum(i - fg_total, 0)
        cb = lax.div(ib, mbsafe)
        jb = ib - cb * mbsafe
        pf = plsc.load_gather(idxfg_v, [jf])
        pb = plsc.load_gather(idxbg_v, [jb])
        src = jnp.where(isfg, cf * P + pf, cb * P + pb)
        gidx_v[pl.ds(k * 16, 16)] = src
        return carry

    lax.fori_loop(0, TOT // 16, gstep, jnp.int32(0))

    copies = []
    for j in range(TOT // 128):
        copies.append(pltpu.async_copy(
            qf_hbm.at[gidx_v.at[pl.ds(j * 128, 128)]],
            vals_v.at[pl.ds(j * 128, 128)], sem))
    for cp in copies:
        cp.wait()
    pltpu.sync_copy(vals_v, packed_hbm.at[pl.ds(base, TOT)])

    @pl.when(wid == 0)
    def _():
        m_v[...] = jnp.broadcast_to(M, (16,)).astype(jnp.int32)
        pltpu.sync_copy(m_v, m_hbm)


def _sc_pack(ql_t, cs, q_flat, P, C):
    info = plsc.get_sparse_core_info()
    NC, NS = info.num_cores, info.num_subcores
    NW = NC * NS
    assert (C * P) % NW == 0 and P % 16 == 0
    TOT = (C * P) // NW
    assert TOT % 128 == 0
    mesh = plsc.VectorSubcoreMesh(core_axis_name="c", subcore_axis_name="s")
    body = functools.partial(_sc_pack_body, P, C, NC, TOT)
    return pl.kernel(
        body,
        compiler_params=pltpu.CompilerParams(needs_layout_passes=False),
        out_type=[
            jax.ShapeDtypeStruct((C * P,), jnp.float32),
            jax.ShapeDtypeStruct((16,), jnp.int32),
        ],
        mesh=mesh,
        scratch_types=[
            pltpu.VMEM((3, P), jnp.int32),
            pltpu.VMEM((3, 16), jnp.int32),
            pltpu.VMEM((P + 16,), jnp.int32),
            pltpu.VMEM((P + 16,), jnp.int32),
            pltpu.VMEM((TOT,), jnp.int32),
            pltpu.VMEM((TOT,), jnp.float32),
            pltpu.VMEM((16,), jnp.int32),
            pltpu.SemaphoreType.DMA,
        ],
    )(ql_t, cs, q_flat)


def _merge22(A, B):
    a0, a1 = A
    b0, b1 = B
    y0 = jnp.maximum(a0, b1)
    y2 = jnp.minimum(a0, b1)
    y1 = jnp.maximum(a1, b0)
    y3 = jnp.minimum(a1, b0)
    return (jnp.maximum(y0, y1), jnp.minimum(y0, y1),
            jnp.maximum(y2, y3), jnp.minimum(y2, y3))


def _merge44_top4(A, B):
    L0 = jnp.maximum(A[0], B[3])
    L1 = jnp.maximum(A[1], B[2])
    L2 = jnp.maximum(A[2], B[1])
    L3 = jnp.maximum(A[3], B[0])
    y0 = jnp.maximum(L0, L2)
    y2 = jnp.minimum(L0, L2)
    y1 = jnp.maximum(L1, L3)
    y3 = jnp.minimum(L1, L3)
    return (jnp.maximum(y0, y1), jnp.minimum(y0, y1),
            jnp.maximum(y2, y3), jnp.minimum(y2, y3))


def _summary4(zp, PW):
    chunks = [zp[:, k * 128:(k + 1) * 128] for k in range(PW // 128)]
    n2 = [(jnp.maximum(a, b), jnp.minimum(a, b))
          for a, b in zip(chunks[0::2], chunks[1::2])]
    n4 = [_merge22(a, b) for a, b in zip(n2[0::2], n2[1::2])]
    while len(n4) > 1:
        n4 = [_merge44_top4(a, b) for a, b in zip(n4[0::2], n4[1::2])]
    return n4[0]


def _tc_body(P, PW, m_sref, s_ref, f_ref, fg_ref, bg_ref):
    M = m_sref[0]
    kf = jnp.float32(_K)
    neg3 = jnp.float32(-3.0)
    ninf = jnp.float32(-jnp.inf)
    s = s_ref[...]
    sn = s / jnp.maximum(
        jnp.sqrt(jnp.sum(s * s, axis=1, keepdims=True)), _EPS)
    BR = sn.shape[0]
    ones_pw = jnp.ones((PW, 1), jnp.float32)
    ones_p = jnp.ones((P, 1), jnp.float32)

    def normcols(fp):
        return fp / jnp.maximum(
            jnp.sqrt(jnp.sum(fp * fp, axis=1, keepdims=True)), _EPS)

    def zpanel(p, is_fg):
        base = pl.multiple_of(p * PW, PW)
        fnp = normcols(f_ref[pl.ds(base, PW), :])
        sp = lax.dot_general(sn, fnp, (((1,), (1,)), ((), ())),
                             preferred_element_type=jnp.float32)
        colp = base + lax.broadcasted_iota(jnp.int32, sp.shape, 1)
        live = (colp < M) if is_fg else (colp >= M)
        return jnp.where(live, sp, neg3)

    def side_sum(is_fg):
        if is_fg:
            plo = jnp.int32(0)
            phi = lax.div(M + (PW - 1), PW)
        else:
            plo = lax.div(M, PW)
            phi = jnp.int32(P // PW)

        def l1(p, carry):
            return _merge44_top4(carry, _summary4(zpanel(p, is_fg), PW))

        init = tuple(jnp.full((BR, 128), ninf, jnp.float32) for _ in range(4))
        a, b, c, d = lax.fori_loop(plo, phi, l1, init)

        out = []
        for _ in range(_K):
            mx = jnp.max(a, axis=1, keepdims=True)
            out.append(mx)
            hit = a == mx
            a = jnp.where(hit, b, a)
            b = jnp.where(hit, c, b)
            c = jnp.where(hit, d, c)
            d = jnp.where(hit, ninf, d)
        vals = jnp.concatenate(out, axis=1)
        tau = vals[:, _K - 1:_K]
        s0 = jnp.sum(vals, axis=1, keepdims=True)

        def l2(p, cnt):
            indf = jnp.where(zpanel(p, is_fg) >= tau, 1.0, 0.0)
            return cnt + lax.dot_general(indf, ones_pw,
                                         (((1,), (0,)), ((), ())),
                                         preferred_element_type=jnp.float32)

        cnt = lax.fori_loop(plo, phi, l2, jnp.zeros((BR, 1), jnp.float32))

        def rowsum(x):
            return lax.dot_general(x, ones_p, (((1,), (0,)), ((), ())),
                                   preferred_element_type=jnp.float32)

        def fallback():
            fn = normcols(f_ref[...])
            sim = lax.dot_general(sn, fn, (((1,), (1,)), ((), ())),
                                  preferred_element_type=jnp.float32)
            col = lax.broadcasted_iota(jnp.int32, sim.shape, 1)
            live = (col < M) if is_fg else (col >= M)
            z = jnp.where(live, sim, neg3)
            ind = z >= tau
            cntf = rowsum(jnp.where(ind, 1.0, 0.0))
            ssum = rowsum(jnp.where(ind, z, 0.0))
            zc = jnp.where(ind, z, jnp.float32(jnp.inf))

            def cond(st):
                _, _, c0 = st
                return jnp.any(c0 > kf)

            def body(st):
                zc, ssum, c0 = st
                active = c0 > kf
                mn = jnp.min(zc, axis=1, keepdims=True)
                eqc = zc == mn
                c_mn = rowsum(jnp.where(eqc, 1.0, 0.0))
                take = jnp.where(active, jnp.minimum(c_mn, c0 - kf), 0.0)
                mn0 = jnp.where(active, mn, 0.0)
                ssum = ssum - mn0 * take
                c0 = c0 - take
                zc = jnp.where(eqc & active, jnp.float32(jnp.inf), zc)
                return zc, ssum, c0

            _, ssum, _ = lax.while_loop(cond, body, (zc, ssum, cntf))
            return ssum

        return lax.cond(jnp.any(cnt != kf), fallback, lambda: s0)[:, 0]

    scale = jnp.float32(1.0 / _K)
    fgm = side_sum(True) * scale
    bgm = side_sum(False) * scale

    zero = jnp.float32(0.0)
    fg_ref[...] = jnp.where(M >= _K, fgm, jnp.where(M > 0, ninf, zero))
    bg_ref[...] = jnp.where(P - M >= _K, bgm,
                            jnp.where(M < P, ninf, zero))


def _tc_score(m, s2d, packed2d, BR=512, PW=1024):
    S, C = s2d.shape
    P = packed2d.shape[0]
    assert S % BR == 0 and P % PW == 0 and PW % 256 == 0
    grid_spec = pltpu.PrefetchScalarGridSpec(
        num_scalar_prefetch=1,
        grid=(S // BR,),
        in_specs=[
            pl.BlockSpec((BR, C), lambda i, m_ref: (i, 0)),
            pl.BlockSpec((P, C), lambda i, m_ref: (0, 0)),
        ],
        out_specs=[
            pl.BlockSpec((BR,), lambda i, m_ref: (i,)),
            pl.BlockSpec((BR,), lambda i, m_ref: (i,)),
        ],
    )
    return pl.pallas_call(
        functools.partial(_tc_body, P, PW),
        grid_spec=grid_spec,
        out_shape=[
            jax.ShapeDtypeStruct((S,), jnp.float32),
            jax.ShapeDtypeStruct((S,), jnp.float32),
        ],
    )(m, s2d, packed2d)


def kernel(query_label, color, q_feat, s_feat):
    C = q_feat.shape[1]
    P = q_feat.shape[2] * q_feat.shape[3]
    out_shape = s_feat.shape[2:]
    S = out_shape[0] * out_shape[1]

    ql_t = query_label.reshape(P, 3).T.astype(jnp.int32)
    cs = jnp.broadcast_to(color.astype(jnp.int32)[:, None], (3, 16))
    q_flat = q_feat.reshape(-1)
    s2d = s_feat.reshape(S, C)

    packed, m = _sc_pack(ql_t, cs, q_flat, P, C)
    packed2d = packed.reshape(P, C)
    fg, bg = _tc_score(m, s2d, packed2d)
    return fg.reshape(out_shape), bg.reshape(out_shape)


# speedup vs baseline: 1.0832x; 1.0832x over previous
import functools

import jax
import jax.numpy as jnp
from jax import lax
from jax.experimental import pallas as pl
from jax.experimental.pallas import tpu as pltpu
from jax.experimental.pallas import tpu_sc as plsc

_K = 20
_EPS = 1e-12


def _sc_pack_body(P, C, NC, TOT,
                  ql_hbm, cs_hbm, qf_hbm, packed_hbm, m_hbm,
                  ql_v, cs_v, idxfg_v, idxbg_v, gidx_v, vals_v, m_v, sem):
    wid = lax.axis_index("s") * NC + lax.axis_index("c")
    pltpu.sync_copy(ql_hbm, ql_v)
    pltpu.sync_copy(cs_hbm, cs_v)
    c0 = cs_v[0, :]
    c1 = cs_v[1, :]
    c2 = cs_v[2, :]
    iota = lax.iota(jnp.int32, 16)

    def mstep(k, carry):
        ofg, obg = carry
        a = ql_v[0, pl.ds(k * 16, 16)]
        b = ql_v[1, pl.ds(k * 16, 16)]
        e = ql_v[2, pl.ds(k * 16, 16)]
        mk = (a == c0) & (b == c1) & (e == c2)
        pvec = k * 16 + iota
        plsc.store_compressed(idxfg_v.at[pl.ds(ofg, 16)], pvec, mask=mk)
        plsc.store_compressed(idxbg_v.at[pl.ds(obg, 16)], pvec,
                              mask=jnp.logical_not(mk))
        cnt = plsc.all_reduce_population_count(mk)[0]
        return (ofg + cnt, obg + (16 - cnt))

    M, Mb = lax.fori_loop(0, P // 16, mstep, (jnp.int32(0), jnp.int32(0)))

    fg_total = C * M
    msafe = jnp.maximum(M, 1)
    mbsafe = jnp.maximum(Mb, 1)
    base = wid * TOT

    def gstep(k, carry):
        i = base + k * 16 + iota
        isfg = i < fg_total
        cf = lax.div(i, msafe)
        jf = i - cf * msafe
        ib = jnp.maximum(i - fg_total, 0)
        cb = lax.div(ib, mbsafe)
        jb = ib - cb * mbsafe
        pf = plsc.load_gather(idxfg_v, [jf])
        pb = plsc.load_gather(idxbg_v, [jb])
        src = jnp.where(isfg, cf * P + pf, cb * P + pb)
        gidx_v[pl.ds(k * 16, 16)] = src
        return carry

    lax.fori_loop(0, TOT // 16, gstep, jnp.int32(0))

    copies = []
    for j in range(TOT // 128):
        copies.append(pltpu.async_copy(
            qf_hbm.at[gidx_v.at[pl.ds(j * 128, 128)]],
            vals_v.at[pl.ds(j * 128, 128)], sem))
    for cp in copies:
        cp.wait()
    pltpu.sync_copy(vals_v, packed_hbm.at[pl.ds(base, TOT)])

    @pl.when(wid == 0)
    def _():
        m_v[...] = jnp.broadcast_to(M, (16,)).astype(jnp.int32)
        pltpu.sync_copy(m_v, m_hbm)


def _sc_pack(ql_t, cs, q_flat, P, C):
    info = plsc.get_sparse_core_info()
    NC, NS = info.num_cores, info.num_subcores
    NW = NC * NS
    assert (C * P) % NW == 0 and P % 16 == 0
    TOT = (C * P) // NW
    assert TOT % 128 == 0
    mesh = plsc.VectorSubcoreMesh(core_axis_name="c", subcore_axis_name="s")
    body = functools.partial(_sc_pack_body, P, C, NC, TOT)
    return pl.kernel(
        body,
        compiler_params=pltpu.CompilerParams(needs_layout_passes=False),
        out_type=[
            jax.ShapeDtypeStruct((C * P,), jnp.float32),
            jax.ShapeDtypeStruct((16,), jnp.int32),
        ],
        mesh=mesh,
        scratch_types=[
            pltpu.VMEM((3, P), jnp.int32),
            pltpu.VMEM((3, 16), jnp.int32),
            pltpu.VMEM((P + 16,), jnp.int32),
            pltpu.VMEM((P + 16,), jnp.int32),
            pltpu.VMEM((TOT,), jnp.int32),
            pltpu.VMEM((TOT,), jnp.float32),
            pltpu.VMEM((16,), jnp.int32),
            pltpu.SemaphoreType.DMA,
        ],
    )(ql_t, cs, q_flat)


def _tkm_vals(x):
    neg = jnp.float32(-jnp.inf)
    out = []
    for _ in range(_K):
        mx = jnp.max(x, axis=1, keepdims=True)
        out.append(mx)
        x = jnp.where(x == mx, neg, x)
    return jnp.concatenate(out, axis=1)


def _topk_sum(z, P):
    kf = jnp.float32(_K)
    ones = jnp.ones((P, 1), jnp.float32)

    def rowsum(x):
        return lax.dot_general(x, ones, (((1,), (0,)), ((), ())),
                               preferred_element_type=jnp.float32)

    def merge22(A, B):
        a0, a1 = A
        b0, b1 = B
        y0 = jnp.maximum(a0, b1)
        y2 = jnp.minimum(a0, b1)
        y1 = jnp.maximum(a1, b0)
        y3 = jnp.minimum(a1, b0)
        return (jnp.maximum(y0, y1), jnp.minimum(y0, y1),
                jnp.maximum(y2, y3), jnp.minimum(y2, y3))

    def merge44_top4(A, B):
        L0 = jnp.maximum(A[0], B[3])
        L1 = jnp.maximum(A[1], B[2])
        L2 = jnp.maximum(A[2], B[1])
        L3 = jnp.maximum(A[3], B[0])
        y0 = jnp.maximum(L0, L2)
        y2 = jnp.minimum(L0, L2)
        y1 = jnp.maximum(L1, L3)
        y3 = jnp.minimum(L1, L3)
        return (jnp.maximum(y0, y1), jnp.minimum(y0, y1),
                jnp.maximum(y2, y3), jnp.minimum(y2, y3))

    chunks = [z[:, k * 128:(k + 1) * 128] for k in range(P // 128)]
    n2 = [(jnp.maximum(a, b), jnp.minimum(a, b))
          for a, b in zip(chunks[0::2], chunks[1::2])]
    n4 = [merge22(a, b) for a, b in zip(n2[0::2], n2[1::2])]
    while len(n4) > 1:
        n4 = [merge44_top4(a, b) for a, b in zip(n4[0::2], n4[1::2])]

    a, b, c, d = n4[0]
    ninf = jnp.float32(-jnp.inf)
    out = []
    for _ in range(_K):
        mx = jnp.max(a, axis=1, keepdims=True)
        out.append(mx)
        hit = a == mx
        a = jnp.where(hit, b, a)
        b = jnp.where(hit, c, b)
        c = jnp.where(hit, d, c)
        d = jnp.where(hit, ninf, d)
    vals = jnp.concatenate(out, axis=1)
    tau = vals[:, _K - 1:_K]
    s0 = jnp.sum(vals, axis=1, keepdims=True)
    ind = z >= tau
    cnt = rowsum(jnp.where(ind, 1.0, 0.0))

    def fallback():
        ssum = rowsum(jnp.where(ind, z, 0.0))
        zc = jnp.where(ind, z, jnp.float32(jnp.inf))

        def cond(st):
            _, _, cnt = st
            return jnp.any(cnt > kf)

        def body(st):
            zc, ssum, cnt = st
            active = cnt > kf
            mn = jnp.min(zc, axis=1, keepdims=True)
            eqc = zc == mn
            c_mn = rowsum(jnp.where(eqc, 1.0, 0.0))
            take = jnp.where(active, jnp.minimum(c_mn, cnt - kf), 0.0)
            mn0 = jnp.where(active, mn, 0.0)
            ssum = ssum - mn0 * take
            cnt = cnt - take
            zc = jnp.where(eqc & active, jnp.float32(jnp.inf), zc)
            return zc, ssum, cnt

        _, ssum, _ = lax.while_loop(cond, body, (zc, ssum, cnt))
        return ssum

    s = lax.cond(jnp.any(cnt != kf), fallback, lambda: s0)
    return s[:, 0]


def _tc_body(P, m_sref, s_ref, f_ref, fg_ref, bg_ref):
    M = m_sref[0]
    neg3 = jnp.float32(-3.0)
    f = f_ref[...]
    fn = f / jnp.maximum(
        jnp.sqrt(jnp.sum(f * f, axis=1, keepdims=True)), _EPS)
    s = s_ref[...]
    sn = s / jnp.maximum(
        jnp.sqrt(jnp.sum(s * s, axis=1, keepdims=True)), _EPS)
    sim = lax.dot_general(sn, fn, (((1,), (1,)), ((), ())),
                          preferred_element_type=jnp.float32)
    col = lax.broadcasted_iota(jnp.int32, sim.shape, 1)
    livefg = col < M
    scale = jnp.float32(1.0 / _K)
    fgm = _topk_sum(jnp.where(livefg, sim, neg3), P) * scale
    bgm = _topk_sum(jnp.where(livefg, neg3, sim), P) * scale

    ninf = jnp.float32(-jnp.inf)
    zero = jnp.float32(0.0)
    fg_ref[...] = jnp.where(M >= _K, fgm, jnp.where(M > 0, ninf, zero))
    bg_ref[...] = jnp.where(P - M >= _K, bgm,
                            jnp.where(M < P, ninf, zero))


def _tc_score(m, s2d, packed2d, BR=512):
    S, C = s2d.shape
    P = packed2d.shape[0]
    assert S % BR == 0 and P % 128 == 0
    grid_spec = pltpu.PrefetchScalarGridSpec(
        num_scalar_prefetch=1,
        grid=(S // BR,),
        in_specs=[
            pl.BlockSpec((BR, C), lambda i, m_ref: (i, 0)),
            pl.BlockSpec((P, C), lambda i, m_ref: (0, 0)),
        ],
        out_specs=[
            pl.BlockSpec((BR,), lambda i, m_ref: (i,)),
            pl.BlockSpec((BR,), lambda i, m_ref: (i,)),
        ],
    )
    return pl.pallas_call(
        functools.partial(_tc_body, P),
        grid_spec=grid_spec,
        out_shape=[
            jax.ShapeDtypeStruct((S,), jnp.float32),
            jax.ShapeDtypeStruct((S,), jnp.float32),
        ],
    )(m, s2d, packed2d)


def kernel(query_label, color, q_feat, s_feat):
    C = q_feat.shape[1]
    P = q_feat.shape[2] * q_feat.shape[3]
    out_shape = s_feat.shape[2:]
    S = out_shape[0] * out_shape[1]

    ql_t = query_label.reshape(P, 3).T.astype(jnp.int32)
    cs = jnp.broadcast_to(color.astype(jnp.int32)[:, None], (3, 16))
    q_flat = q_feat.reshape(-1)
    s2d = s_feat.reshape(S, C)

    packed, m = _sc_pack(ql_t, cs, q_flat, P, C)
    packed2d = packed.reshape(P, C)
    fg, bg = _tc_score(m, s2d, packed2d)
    return fg.reshape(out_shape), bg.reshape(out_shape)
